# baseline (speedup 1.0000x reference)
import functools

import jax
import jax.numpy as jnp
from jax import lax
from jax.experimental import pallas as pl
from jax.experimental.pallas import tpu as pltpu
from jax.experimental.pallas import tpu_sc as plsc

_BATCH = 16384
_K = 16
_NW = 32
_BPW = _BATCH // _NW
_CHUNK = 128
_NCHUNK = _BPW // _CHUNK


def _mf_body(w_hbm, h_hbm, uidx_hbm, vidx_hbm, out_hbm,
             uidx_v, vidx_v, urows_v, vrows_v, out_v, usem, vsem):
  wid = lax.axis_index("s") * 2 + lax.axis_index("c")
  base = wid * _BPW

  pltpu.sync_copy(uidx_hbm.at[pl.ds(wid * _NCHUNK, _NCHUNK)], uidx_v)
  pltpu.sync_copy(vidx_hbm.at[pl.ds(wid * _NCHUNK, _NCHUNK)], vidx_v)

  copies = []
  for j in range(_NCHUNK):
    copies.append(pltpu.async_copy(
        w_hbm.at[uidx_v.at[j]], urows_v.at[pl.ds(j * _CHUNK, _CHUNK)], usem))
    copies.append(pltpu.async_copy(
        h_hbm.at[vidx_v.at[j]], vrows_v.at[pl.ds(j * _CHUNK, _CHUNK)], vsem))
  for c in copies:
    c.wait()

  lanes = lax.iota(jnp.int32, _K)

  def group(g, carry):
    row0 = pl.multiple_of(g * _K, _K)
    rows = row0 + lanes
    acc = jnp.zeros((_K,), jnp.float32)
    for j in range(_K):
      cols = jnp.full((_K,), j, jnp.int32)
      u = plsc.load_gather(urows_v, [rows, cols])
      v = plsc.load_gather(vrows_v, [rows, cols])
      acc = acc + u * v
    out_v[pl.ds(row0, _K)] = acc
    return carry

  lax.fori_loop(0, _BPW // _K, group, 0)

  pltpu.sync_copy(out_v, out_hbm.at[pl.ds(base, _BPW)])


@jax.jit
def kernel(x, W, H):
  uidx = x[:, 0].astype(jnp.int32).reshape(_NW * _NCHUNK, _CHUNK)
  vidx = x[:, 1].astype(jnp.int32).reshape(_NW * _NCHUNK, _CHUNK)

  mf = pl.kernel(
      _mf_body,
      out_type=jax.ShapeDtypeStruct((_BATCH,), jnp.float32),
      mesh=plsc.VectorSubcoreMesh(core_axis_name="c", subcore_axis_name="s",
                                  num_cores=2, num_subcores=16),
      compiler_params=pltpu.CompilerParams(
          needs_layout_passes=False, use_tc_tiling_on_sc=False),
      scratch_types=[
          pltpu.VMEM((_NCHUNK, _CHUNK), jnp.int32),
          pltpu.VMEM((_NCHUNK, _CHUNK), jnp.int32),
          pltpu.VMEM((_BPW, _K), jnp.float32),
          pltpu.VMEM((_BPW, _K), jnp.float32),
          pltpu.VMEM((_BPW,), jnp.float32),
          pltpu.SemaphoreType.DMA,
          pltpu.SemaphoreType.DMA,
      ],
  )
  return mf(W, H, uidx, vidx)
